# Initial kernel scaffold; baseline (speedup 1.0000x reference)
import jax
import jax.numpy as jnp
from jax.experimental import pallas as pl


def kernel(features, edge_index, W1, b1, W2, b2, W3, b3, Wp, bp):
    raise NotImplementedError("write your pallas kernel here")


# speedup vs baseline: 5.3762x; 5.3762x over previous
import functools

import jax
import jax.numpy as jnp
from jax import lax
from jax.experimental import pallas as pl
from jax.experimental.pallas import tpu as pltpu
from jax.experimental.pallas import tpu_sc as plsc

_NC = 2
_NS = 16
_TILES = _NC * _NS
_DEGW = 16
_G = 128


def _inv_sqrt(d):
    safe = jnp.where(d > 0, d, 1.0)
    return jnp.where(d > 0, lax.rsqrt(safe), 0.0)


def _mesh():
    return plsc.VectorSubcoreMesh(core_axis_name="c", subcore_axis_name="s",
                                  num_cores=_NC, num_subcores=_NS)


def _make_agg_kernel(npad, d, ng):
    rows_per_tile = npad // _NS
    zchunk = _G
    nz = rows_per_tile // zchunk

    cg = 8
    assert ng % cg == 0

    @functools.partial(
        pl.kernel,
        mesh=_mesh(),
        out_type=jax.ShapeDtypeStruct((_NC, npad, d), jnp.float32),
        scratch_types=[
            pltpu.VMEM_SHARED((npad, d), jnp.float32),
            pltpu.VMEM((cg, _G), jnp.int32),
            pltpu.VMEM((cg, _G), jnp.int32),
            pltpu.VMEM((_G, d), jnp.float32),
            pltpu.SemaphoreType.DMA,
        ],
    )
    def agg_kernel(z_hbm, src_hbm, dst_hbm, zeros_hbm, out_hbm,
                   acc, idx_s, idx_d, rows_v, sem):
        c = lax.axis_index("c")
        s = lax.axis_index("s")
        t = c * _NS + s
        base = s * rows_per_tile
        pltpu.sync_copy(zeros_hbm, rows_v)
        for k in range(nz):
            pltpu.sync_copy(rows_v, acc.at[pl.ds(base + k * zchunk, zchunk)])
        plsc.subcore_barrier()

        def chunk_body(cc, carry):
            pltpu.sync_copy(src_hbm.at[t, pl.ds(cc * cg, cg)], idx_s)
            pltpu.sync_copy(dst_hbm.at[t, pl.ds(cc * cg, cg)], idx_d)

            def body(j, carry2):
                pltpu.async_copy(z_hbm.at[idx_s.at[j]], rows_v, sem).wait()
                pltpu.sync_copy(rows_v, acc.at[idx_d.at[j]], add=True)
                return carry2

            return lax.fori_loop(0, cg, body, carry)

        lax.fori_loop(0, ng // cg, chunk_body, 0)
        plsc.subcore_barrier()
        for k in range(nz):
            sl = pl.ds(base + k * zchunk, zchunk)
            pltpu.sync_copy(acc.at[sl], rows_v)
            pltpu.sync_copy(rows_v, out_hbm.at[c, sl])

    return agg_kernel


def _mm_first_body(x_ref, w_ref, dego_ref, o_ref):
    z = jnp.dot(x_ref[...], w_ref[...], preferred_element_type=jnp.float32)
    o_ref[...] = z * _inv_sqrt(dego_ref[...])


def _mm_mid_body(p_ref, degi_ref, b_ref, w_ref, dego_ref, o_ref):
    h = (p_ref[0] + p_ref[1]) * _inv_sqrt(degi_ref[...]) + b_ref[...]
    h = jnp.maximum(h, 0.0)
    z = jnp.dot(h, w_ref[...], preferred_element_type=jnp.float32)
    o_ref[...] = z * _inv_sqrt(dego_ref[...])


def _make_readout_body(n_real):
    def _readout_body(p_ref, degi_ref, b_ref, wp_ref, bp_ref, o_ref):
        npad = p_ref.shape[1]
        h = (p_ref[0] + p_ref[1]) * _inv_sqrt(degi_ref[...]) + b_ref[...]
        h = jnp.maximum(h, 0.0)
        row = lax.broadcasted_iota(jnp.int32, (npad, 1), 0)
        h = jnp.where(row < n_real, h, 0.0)
        r = jnp.sum(h, axis=0, keepdims=True) * (1.0 / n_real)
        o_ref[...] = jnp.dot(r, wp_ref[...],
                             preferred_element_type=jnp.float32) + bp_ref[...]
    return _readout_body


def kernel(features, edge_index, W1, b1, W2, b2, W3, b3, Wp, bp):
    n, d = features.shape
    e = edge_index.shape[1]
    npad = ((n + _NS * 128 - 1) // (_NS * 128)) * (_NS * 128)
    ept = e // _TILES
    assert ept * _TILES == e
    ng = ((ept + _G - 1) // _G + 7) // 8 * 8
    slots = ng * _G
    padcnt = slots - ept

    src2 = edge_index[0].reshape(_TILES, ept)
    dst2 = edge_index[1].reshape(_TILES, ept)
    padv = (n + (jnp.arange(padcnt, dtype=jnp.int32) % (npad - n)))
    padv = jnp.broadcast_to(padv, (_TILES, padcnt))
    src3 = jnp.concatenate([src2, padv], axis=1).reshape(_TILES, ng, _G)
    dst3 = jnp.concatenate([dst2, padv], axis=1).reshape(_TILES, ng, _G)

    zerosd = jnp.zeros((128, d), jnp.float32)
    onesnd = jnp.ones((npad, d), jnp.float32)
    xpad = jnp.zeros((npad, d), features.dtype).at[:n].set(features)

    agg = _make_agg_kernel(npad, d, ng)

    degi_p = agg(onesnd, src3, dst3, zerosd)
    dego_p = agg(onesnd, dst3, src3, zerosd)
    degi = degi_p[0, :, :1] + degi_p[1, :, :1]
    dego = dego_p[0, :, :1] + dego_p[1, :, :1]
    mm_first = pl.pallas_call(
        _mm_first_body, out_shape=jax.ShapeDtypeStruct((npad, d), jnp.float32))
    mm_mid = pl.pallas_call(
        _mm_mid_body, out_shape=jax.ShapeDtypeStruct((npad, d), jnp.float32))
    readout = pl.pallas_call(
        _make_readout_body(n),
        out_shape=jax.ShapeDtypeStruct((1, Wp.shape[1]), jnp.float32))

    b1r, b2r, b3r = b1.reshape(1, d), b2.reshape(1, d), b3.reshape(1, d)
    bpr = bp.reshape(1, -1)

    z1 = mm_first(xpad, W1, dego)
    p1 = agg(z1, src3, dst3, zerosd)
    z2 = mm_mid(p1, degi, b1r, W2, dego)
    p2 = agg(z2, src3, dst3, zerosd)
    z3 = mm_mid(p2, degi, b2r, W3, dego)
    p3 = agg(z3, src3, dst3, zerosd)
    return readout(p3, degi, b3r, Wp, bpr)


# speedup vs baseline: 8.3464x; 1.5525x over previous
import functools

import jax
import jax.numpy as jnp
from jax import lax
from jax.experimental import pallas as pl
from jax.experimental.pallas import tpu as pltpu
from jax.experimental.pallas import tpu_sc as plsc

_NC = 2
_NS = 16
_TILES = _NC * _NS
_DEGW = 16
_G = 128


def _inv_sqrt(d):
    safe = jnp.where(d > 0, d, 1.0)
    return jnp.where(d > 0, lax.rsqrt(safe), 0.0)


def _mesh():
    return plsc.VectorSubcoreMesh(core_axis_name="c", subcore_axis_name="s",
                                  num_cores=_NC, num_subcores=_NS)


def _make_agg_kernel(npad, d, ng):
    rows_per_tile = npad // _NS
    zchunk = _G
    nz = rows_per_tile // zchunk

    cg = 16
    assert ng % cg == 0

    @functools.partial(
        pl.kernel,
        mesh=_mesh(),
        out_type=jax.ShapeDtypeStruct((_NC, npad, d), jnp.float32),
        scratch_types=[
            pltpu.VMEM_SHARED((npad, d), jnp.float32),
            pltpu.VMEM((cg, _G), jnp.int32),
            pltpu.VMEM((cg, _G), jnp.int32),
            pltpu.VMEM((_G, d), jnp.float32),
            pltpu.VMEM((_G, d), jnp.float32),
            pltpu.SemaphoreType.DMA,
            pltpu.SemaphoreType.DMA,
        ],
    )
    def agg_kernel(z_hbm, src_hbm, dst_hbm, zeros_hbm, out_hbm,
                   acc, idx_s, idx_d, rows_a, rows_b, sem_a, sem_b):
        c = lax.axis_index("c")
        s = lax.axis_index("s")
        t = c * _NS + s
        base = s * rows_per_tile
        pltpu.sync_copy(zeros_hbm, rows_a)
        for k in range(nz):
            pltpu.sync_copy(rows_a, acc.at[pl.ds(base + k * zchunk, zchunk)])
        plsc.subcore_barrier()

        bufs = (rows_a, rows_b)
        sems = (sem_a, sem_b)

        def chunk_body(cc, carry):
            pltpu.sync_copy(src_hbm.at[t, pl.ds(cc * cg, cg)], idx_s)
            pltpu.sync_copy(dst_hbm.at[t, pl.ds(cc * cg, cg)], idx_d)
            pend = pltpu.async_copy(z_hbm.at[idx_s.at[0]], bufs[0], sems[0])
            for j in range(cg):
                pend.wait()
                if j + 1 < cg:
                    pend = pltpu.async_copy(z_hbm.at[idx_s.at[j + 1]],
                                            bufs[(j + 1) % 2], sems[(j + 1) % 2])
                pltpu.sync_copy(bufs[j % 2], acc.at[idx_d.at[j]], add=True)
            return carry

        lax.fori_loop(0, ng // cg, chunk_body, 0)
        plsc.subcore_barrier()
        for k in range(nz):
            sl = pl.ds(base + k * zchunk, zchunk)
            pltpu.sync_copy(acc.at[sl], rows_a)
            pltpu.sync_copy(rows_a, out_hbm.at[c, sl])

    return agg_kernel


def _make_deg_kernel(npad, d, ng):
    rows_per_tile = npad // _NS
    zchunk = _G
    nz = rows_per_tile // zchunk
    cg = 16
    assert ng % cg == 0

    @functools.partial(
        pl.kernel,
        mesh=_mesh(),
        out_type=jax.ShapeDtypeStruct((_NC, npad, d), jnp.float32),
        scratch_types=[
            pltpu.VMEM_SHARED((npad, d), jnp.float32),
            pltpu.VMEM((cg, _G), jnp.int32),
            pltpu.VMEM((_G, d), jnp.float32),
            pltpu.VMEM((_G, d), jnp.float32),
            pltpu.SemaphoreType.DMA,
        ],
    )
    def deg_kernel(ids_hbm, ones_hbm, zeros_hbm, out_hbm,
                   acc, idx_v, ones_v, stage, sem):
        c = lax.axis_index("c")
        s = lax.axis_index("s")
        t = c * _NS + s
        base = s * rows_per_tile
        pltpu.sync_copy(ones_hbm, ones_v)
        pltpu.sync_copy(zeros_hbm, stage)
        for k in range(nz):
            pltpu.sync_copy(stage, acc.at[pl.ds(base + k * zchunk, zchunk)])
        plsc.subcore_barrier()

        def chunk_body(cc, carry):
            pltpu.sync_copy(ids_hbm.at[t, pl.ds(cc * cg, cg)], idx_v)
            pends = [pltpu.async_copy(ones_v, acc.at[idx_v.at[j]], sem,
                                      add=True) for j in range(cg)]
            for p in pends:
                p.wait()
            return carry

        lax.fori_loop(0, ng // cg, chunk_body, 0)
        plsc.subcore_barrier()
        for k in range(nz):
            sl = pl.ds(base + k * zchunk, zchunk)
            pltpu.sync_copy(acc.at[sl], stage)
            pltpu.sync_copy(stage, out_hbm.at[c, sl])

    return deg_kernel


def _mm_first_body(x_ref, w_ref, dego_ref, o_ref):
    z = jnp.dot(x_ref[...], w_ref[...], preferred_element_type=jnp.float32)
    o_ref[...] = z * _inv_sqrt(dego_ref[...])


def _mm_mid_body(p_ref, degi_ref, b_ref, w_ref, dego_ref, o_ref):
    h = (p_ref[0] + p_ref[1]) * _inv_sqrt(degi_ref[...]) + b_ref[...]
    h = jnp.maximum(h, 0.0)
    z = jnp.dot(h, w_ref[...], preferred_element_type=jnp.float32)
    o_ref[...] = z * _inv_sqrt(dego_ref[...])


def _make_readout_body(n_real):
    def _readout_body(p_ref, degi_ref, b_ref, wp_ref, bp_ref, o_ref):
        npad = p_ref.shape[1]
        h = (p_ref[0] + p_ref[1]) * _inv_sqrt(degi_ref[...]) + b_ref[...]
        h = jnp.maximum(h, 0.0)
        row = lax.broadcasted_iota(jnp.int32, (npad, 1), 0)
        h = jnp.where(row < n_real, h, 0.0)
        r = jnp.sum(h, axis=0, keepdims=True) * (1.0 / n_real)
        o_ref[...] = jnp.dot(r, wp_ref[...],
                             preferred_element_type=jnp.float32) + bp_ref[...]
    return _readout_body


def kernel(features, edge_index, W1, b1, W2, b2, W3, b3, Wp, bp):
    n, d = features.shape
    e = edge_index.shape[1]
    npad = ((n + _NS * 128 - 1) // (_NS * 128)) * (_NS * 128)
    ept = e // _TILES
    assert ept * _TILES == e
    ng = ((ept + _G - 1) // _G + 7) // 8 * 8
    slots = ng * _G
    padcnt = slots - ept

    src2 = edge_index[0].reshape(_TILES, ept)
    dst2 = edge_index[1].reshape(_TILES, ept)
    padv = (n + (jnp.arange(padcnt, dtype=jnp.int32) % (npad - n)))
    padv = jnp.broadcast_to(padv, (_TILES, padcnt))
    src3 = jnp.concatenate([src2, padv], axis=1).reshape(_TILES, ng, _G)
    dst3 = jnp.concatenate([dst2, padv], axis=1).reshape(_TILES, ng, _G)

    zerosd = jnp.zeros((128, d), jnp.float32)
    onesd = jnp.ones((_G, d), jnp.float32)
    xpad = jnp.zeros((npad, d), features.dtype).at[:n].set(features)

    agg = _make_agg_kernel(npad, d, ng)
    deg = _make_deg_kernel(npad, d, ng)

    degi_p = deg(dst3, onesd, zerosd)
    dego_p = deg(src3, onesd, zerosd)
    degi = degi_p[0, :, :1] + degi_p[1, :, :1]
    dego = dego_p[0, :, :1] + dego_p[1, :, :1]
    mm_first = pl.pallas_call(
        _mm_first_body, out_shape=jax.ShapeDtypeStruct((npad, d), jnp.float32))
    mm_mid = pl.pallas_call(
        _mm_mid_body, out_shape=jax.ShapeDtypeStruct((npad, d), jnp.float32))
    readout = pl.pallas_call(
        _make_readout_body(n),
        out_shape=jax.ShapeDtypeStruct((1, Wp.shape[1]), jnp.float32))

    b1r, b2r, b3r = b1.reshape(1, d), b2.reshape(1, d), b3.reshape(1, d)
    bpr = bp.reshape(1, -1)

    z1 = mm_first(xpad, W1, dego)
    p1 = agg(z1, src3, dst3, zerosd)
    z2 = mm_mid(p1, degi, b1r, W2, dego)
    p2 = agg(z2, src3, dst3, zerosd)
    z3 = mm_mid(p2, degi, b2r, W3, dego)
    p3 = agg(z3, src3, dst3, zerosd)
    return readout(p3, degi, b3r, Wp, bpr)


# speedup vs baseline: 8.3477x; 1.0002x over previous
import functools

import jax
import jax.numpy as jnp
from jax import lax
from jax.experimental import pallas as pl
from jax.experimental.pallas import tpu as pltpu
from jax.experimental.pallas import tpu_sc as plsc

_NC = 2
_NS = 16
_TILES = _NC * _NS
_DEGW = 16
_G = 128


def _inv_sqrt(d):
    safe = jnp.where(d > 0, d, 1.0)
    return jnp.where(d > 0, lax.rsqrt(safe), 0.0)


def _mesh():
    return plsc.VectorSubcoreMesh(core_axis_name="c", subcore_axis_name="s",
                                  num_cores=_NC, num_subcores=_NS)


def _make_agg_kernel(npad, d, ng):
    rows_per_tile = npad // _NS
    zchunk = _G
    nz = rows_per_tile // zchunk

    cg = 16
    assert ng % cg == 0

    @functools.partial(
        pl.kernel,
        mesh=_mesh(),
        out_type=jax.ShapeDtypeStruct((_NC, npad, d), jnp.float32),
        scratch_types=[
            pltpu.VMEM_SHARED((npad, d), jnp.float32),
            pltpu.VMEM((cg, _G), jnp.int32),
            pltpu.VMEM((cg, _G), jnp.int32),
            pltpu.VMEM((_G, d), jnp.float32),
            pltpu.VMEM((_G, d), jnp.float32),
            pltpu.SemaphoreType.DMA,
            pltpu.SemaphoreType.DMA,
            pltpu.SemaphoreType.DMA,
            pltpu.SemaphoreType.DMA,
        ],
    )
    def agg_kernel(z_hbm, src_hbm, dst_hbm, zeros_hbm, out_hbm,
                   acc, idx_s, idx_d, rows_a, rows_b,
                   gsem_a, gsem_b, ssem_a, ssem_b):
        c = lax.axis_index("c")
        s = lax.axis_index("s")
        t = c * _NS + s
        base = s * rows_per_tile
        pltpu.sync_copy(zeros_hbm, rows_a)
        for k in range(nz):
            pltpu.sync_copy(rows_a, acc.at[pl.ds(base + k * zchunk, zchunk)])
        plsc.subcore_barrier()

        bufs = (rows_a, rows_b)
        gsems = (gsem_a, gsem_b)
        ssems = (ssem_a, ssem_b)

        def chunk_body(cc, carry):
            pltpu.sync_copy(src_hbm.at[t, pl.ds(cc * cg, cg)], idx_s)
            pltpu.sync_copy(dst_hbm.at[t, pl.ds(cc * cg, cg)], idx_d)
            pend_g = [pltpu.async_copy(z_hbm.at[idx_s.at[0]], bufs[0],
                                       gsems[0]), None]
            pend_s = [None, None]
            for j in range(cg):
                p = j % 2
                q = 1 - p
                pend_g[p].wait()
                if pend_s[q] is not None:
                    pend_s[q].wait()
                if j + 1 < cg:
                    pend_g[q] = pltpu.async_copy(z_hbm.at[idx_s.at[j + 1]],
                                                 bufs[q], gsems[q])
                pend_s[p] = pltpu.async_copy(bufs[p], acc.at[idx_d.at[j]],
                                             ssems[p], add=True)
            pend_s[(cg - 1) % 2].wait()
            return carry

        lax.fori_loop(0, ng // cg, chunk_body, 0)
        plsc.subcore_barrier()
        for k in range(nz):
            sl = pl.ds(base + k * zchunk, zchunk)
            pltpu.sync_copy(acc.at[sl], rows_a)
            pltpu.sync_copy(rows_a, out_hbm.at[c, sl])

    return agg_kernel


def _make_deg_kernel(npad, d, ng):
    rows_per_tile = npad // _NS
    zchunk = _G
    nz = rows_per_tile // zchunk
    cg = 16
    assert ng % cg == 0

    @functools.partial(
        pl.kernel,
        mesh=_mesh(),
        out_type=jax.ShapeDtypeStruct((_NC, npad, d), jnp.float32),
        scratch_types=[
            pltpu.VMEM_SHARED((npad, d), jnp.float32),
            pltpu.VMEM((cg, _G), jnp.int32),
            pltpu.VMEM((_G, d), jnp.float32),
            pltpu.VMEM((_G, d), jnp.float32),
            pltpu.SemaphoreType.DMA,
        ],
    )
    def deg_kernel(ids_hbm, ones_hbm, zeros_hbm, out_hbm,
                   acc, idx_v, ones_v, stage, sem):
        c = lax.axis_index("c")
        s = lax.axis_index("s")
        t = c * _NS + s
        base = s * rows_per_tile
        pltpu.sync_copy(ones_hbm, ones_v)
        pltpu.sync_copy(zeros_hbm, stage)
        for k in range(nz):
            pltpu.sync_copy(stage, acc.at[pl.ds(base + k * zchunk, zchunk)])
        plsc.subcore_barrier()

        def chunk_body(cc, carry):
            pltpu.sync_copy(ids_hbm.at[t, pl.ds(cc * cg, cg)], idx_v)
            pends = [pltpu.async_copy(ones_v, acc.at[idx_v.at[j]], sem,
                                      add=True) for j in range(cg)]
            for p in pends:
                p.wait()
            return carry

        lax.fori_loop(0, ng // cg, chunk_body, 0)
        plsc.subcore_barrier()
        for k in range(nz):
            sl = pl.ds(base + k * zchunk, zchunk)
            pltpu.sync_copy(acc.at[sl], stage)
            pltpu.sync_copy(stage, out_hbm.at[c, sl])

    return deg_kernel


def _mm_first_body(x_ref, w_ref, dego_ref, o_ref):
    z = jnp.dot(x_ref[...], w_ref[...], preferred_element_type=jnp.float32)
    o_ref[...] = z * _inv_sqrt(dego_ref[...])


def _mm_mid_body(p_ref, degi_ref, b_ref, w_ref, dego_ref, o_ref):
    h = (p_ref[0] + p_ref[1]) * _inv_sqrt(degi_ref[...]) + b_ref[...]
    h = jnp.maximum(h, 0.0)
    z = jnp.dot(h, w_ref[...], preferred_element_type=jnp.float32)
    o_ref[...] = z * _inv_sqrt(dego_ref[...])


def _make_readout_body(n_real):
    def _readout_body(p_ref, degi_ref, b_ref, wp_ref, bp_ref, o_ref):
        npad = p_ref.shape[1]
        h = (p_ref[0] + p_ref[1]) * _inv_sqrt(degi_ref[...]) + b_ref[...]
        h = jnp.maximum(h, 0.0)
        row = lax.broadcasted_iota(jnp.int32, (npad, 1), 0)
        h = jnp.where(row < n_real, h, 0.0)
        r = jnp.sum(h, axis=0, keepdims=True) * (1.0 / n_real)
        o_ref[...] = jnp.dot(r, wp_ref[...],
                             preferred_element_type=jnp.float32) + bp_ref[...]
    return _readout_body


def kernel(features, edge_index, W1, b1, W2, b2, W3, b3, Wp, bp):
    n, d = features.shape
    e = edge_index.shape[1]
    npad = ((n + _NS * 128 - 1) // (_NS * 128)) * (_NS * 128)
    ept = e // _TILES
    assert ept * _TILES == e
    ng = ((ept + _G - 1) // _G + 7) // 8 * 8
    slots = ng * _G
    padcnt = slots - ept

    src2 = edge_index[0].reshape(_TILES, ept)
    dst2 = edge_index[1].reshape(_TILES, ept)
    padv = (n + (jnp.arange(padcnt, dtype=jnp.int32) % (npad - n)))
    padv = jnp.broadcast_to(padv, (_TILES, padcnt))
    src3 = jnp.concatenate([src2, padv], axis=1).reshape(_TILES, ng, _G)
    dst3 = jnp.concatenate([dst2, padv], axis=1).reshape(_TILES, ng, _G)

    zerosd = jnp.zeros((128, d), jnp.float32)
    onesd = jnp.ones((_G, d), jnp.float32)
    xpad = jnp.zeros((npad, d), features.dtype).at[:n].set(features)

    agg = _make_agg_kernel(npad, d, ng)
    deg = _make_deg_kernel(npad, d, ng)

    degi_p = deg(dst3, onesd, zerosd)
    dego_p = deg(src3, onesd, zerosd)
    degi = degi_p[0, :, :1] + degi_p[1, :, :1]
    dego = dego_p[0, :, :1] + dego_p[1, :, :1]
    mm_first = pl.pallas_call(
        _mm_first_body, out_shape=jax.ShapeDtypeStruct((npad, d), jnp.float32))
    mm_mid = pl.pallas_call(
        _mm_mid_body, out_shape=jax.ShapeDtypeStruct((npad, d), jnp.float32))
    readout = pl.pallas_call(
        _make_readout_body(n),
        out_shape=jax.ShapeDtypeStruct((1, Wp.shape[1]), jnp.float32))

    b1r, b2r, b3r = b1.reshape(1, d), b2.reshape(1, d), b3.reshape(1, d)
    bpr = bp.reshape(1, -1)

    z1 = mm_first(xpad, W1, dego)
    p1 = agg(z1, src3, dst3, zerosd)
    z2 = mm_mid(p1, degi, b1r, W2, dego)
    p2 = agg(z2, src3, dst3, zerosd)
    z3 = mm_mid(p2, degi, b2r, W3, dego)
    p3 = agg(z3, src3, dst3, zerosd)
    return readout(p3, degi, b3r, Wp, bpr)


# speedup vs baseline: 9.8005x; 1.1740x over previous
import functools

import jax
import jax.numpy as jnp
from jax import lax
from jax.experimental import pallas as pl
from jax.experimental.pallas import tpu as pltpu
from jax.experimental.pallas import tpu_sc as plsc

_NC = 2
_NS = 16
_TILES = _NC * _NS
_DEGW = 16
_G = 128


def _inv_sqrt(d):
    safe = jnp.where(d > 0, d, 1.0)
    return jnp.where(d > 0, lax.rsqrt(safe), 0.0)


def _mesh():
    return plsc.VectorSubcoreMesh(core_axis_name="c", subcore_axis_name="s",
                                  num_cores=_NC, num_subcores=_NS)


def _make_agg_kernel(npad, d, ng):
    rows_per_tile = npad // _NS
    zchunk = _G
    nz = rows_per_tile // zchunk

    cg = 16
    assert ng % cg == 0

    @functools.partial(
        pl.kernel,
        mesh=_mesh(),
        out_type=jax.ShapeDtypeStruct((_NC, npad, d), jnp.float32),
        scratch_types=[
            pltpu.VMEM_SHARED((npad, d), jnp.float32),
            pltpu.VMEM((cg, _G), jnp.int32),
            pltpu.VMEM((cg, _G), jnp.int32),
            pltpu.VMEM((_G, d), jnp.float32),
            pltpu.VMEM((_G, d), jnp.float32),
            pltpu.SemaphoreType.DMA,
            pltpu.SemaphoreType.DMA,
            pltpu.SemaphoreType.DMA,
            pltpu.SemaphoreType.DMA,
        ],
    )
    def agg_kernel(z_hbm, src_hbm, dst_hbm, zeros_hbm, out_hbm,
                   acc, idx_s, idx_d, rows_a, rows_b,
                   gsem_a, gsem_b, ssem_a, ssem_b):
        c = lax.axis_index("c")
        s = lax.axis_index("s")
        t = c * _NS + s
        base = s * rows_per_tile
        pltpu.sync_copy(zeros_hbm, rows_a)
        for k in range(nz):
            pltpu.sync_copy(rows_a, acc.at[pl.ds(base + k * zchunk, zchunk)])
        plsc.subcore_barrier()

        bufs = (rows_a, rows_b)
        gsems = (gsem_a, gsem_b)
        ssems = (ssem_a, ssem_b)

        def chunk_body(cc, carry):
            pltpu.sync_copy(src_hbm.at[t, pl.ds(cc * cg, cg)], idx_s)
            pltpu.sync_copy(dst_hbm.at[t, pl.ds(cc * cg, cg)], idx_d)
            pend_g = [pltpu.async_copy(z_hbm.at[idx_s.at[0]], bufs[0],
                                       gsems[0]), None]
            pend_s = [None, None]
            for j in range(cg):
                p = j % 2
                q = 1 - p
                pend_g[p].wait()
                if pend_s[q] is not None:
                    pend_s[q].wait()
                if j + 1 < cg:
                    pend_g[q] = pltpu.async_copy(z_hbm.at[idx_s.at[j + 1]],
                                                 bufs[q], gsems[q])
                pend_s[p] = pltpu.async_copy(bufs[p], acc.at[idx_d.at[j]],
                                             ssems[p], add=True)
            pend_s[(cg - 1) % 2].wait()
            return carry

        lax.fori_loop(0, ng // cg, chunk_body, 0)
        plsc.subcore_barrier()
        for k in range(nz):
            sl = pl.ds(base + k * zchunk, zchunk)
            pltpu.sync_copy(acc.at[sl], rows_a)
            pltpu.sync_copy(rows_a, out_hbm.at[c, sl])

    return agg_kernel


def _make_deg_kernel(npad, ng):
    rows_per_tile = npad // _NS
    zchunk = _G
    nz = rows_per_tile // zchunk
    cg = 16
    assert ng % cg == 0

    @functools.partial(
        pl.kernel,
        mesh=_mesh(),
        out_type=jax.ShapeDtypeStruct((_NC, npad, _DEGW), jnp.float32),
        scratch_types=[
            pltpu.VMEM_SHARED((npad, _DEGW), jnp.float32),
            pltpu.VMEM((cg, _G), jnp.int32),
            pltpu.VMEM((_G, _DEGW), jnp.float32),
            pltpu.VMEM((_G, _DEGW), jnp.float32),
            pltpu.SemaphoreType.DMA,
        ],
    )
    def deg_kernel(ids_hbm, ones_hbm, zeros_hbm, out_hbm,
                   acc, idx_v, ones_v, stage, sem):
        c = lax.axis_index("c")
        s = lax.axis_index("s")
        t = c * _NS + s
        base = s * rows_per_tile
        pltpu.sync_copy(ones_hbm, ones_v)
        pltpu.sync_copy(zeros_hbm, stage)
        for k in range(nz):
            pltpu.sync_copy(stage, acc.at[pl.ds(base + k * zchunk, zchunk)])
        plsc.subcore_barrier()

        def chunk_body(cc, carry):
            pltpu.sync_copy(ids_hbm.at[t, pl.ds(cc * cg, cg)], idx_v)
            pends = [pltpu.async_copy(ones_v, acc.at[idx_v.at[j]], sem,
                                      add=True) for j in range(cg)]
            for p in pends:
                p.wait()
            return carry

        lax.fori_loop(0, ng // cg, chunk_body, 0)
        plsc.subcore_barrier()
        for k in range(nz):
            sl = pl.ds(base + k * zchunk, zchunk)
            pltpu.sync_copy(acc.at[sl], stage)
            pltpu.sync_copy(stage, out_hbm.at[c, sl])

    return deg_kernel


def _mm_first_body(x_ref, w_ref, dego_ref, o_ref):
    z = jnp.dot(x_ref[...], w_ref[...], preferred_element_type=jnp.float32)
    o_ref[...] = z * _inv_sqrt(dego_ref[...])


def _mm_mid_body(p_ref, degi_ref, b_ref, w_ref, dego_ref, o_ref):
    h = (p_ref[0] + p_ref[1]) * _inv_sqrt(degi_ref[...]) + b_ref[...]
    h = jnp.maximum(h, 0.0)
    z = jnp.dot(h, w_ref[...], preferred_element_type=jnp.float32)
    o_ref[...] = z * _inv_sqrt(dego_ref[...])


def _make_readout_body(n_real):
    def _readout_body(p_ref, degi_ref, b_ref, wp_ref, bp_ref, o_ref):
        npad = p_ref.shape[1]
        h = (p_ref[0] + p_ref[1]) * _inv_sqrt(degi_ref[...]) + b_ref[...]
        h = jnp.maximum(h, 0.0)
        row = lax.broadcasted_iota(jnp.int32, (npad, 1), 0)
        h = jnp.where(row < n_real, h, 0.0)
        r = jnp.sum(h, axis=0, keepdims=True) * (1.0 / n_real)
        o_ref[...] = jnp.dot(r, wp_ref[...],
                             preferred_element_type=jnp.float32) + bp_ref[...]
    return _readout_body


def kernel(features, edge_index, W1, b1, W2, b2, W3, b3, Wp, bp):
    n, d = features.shape
    e = edge_index.shape[1]
    npad = ((n + _NS * 128 - 1) // (_NS * 128)) * (_NS * 128)
    ept = e // _TILES
    assert ept * _TILES == e
    ng = ((ept + _G - 1) // _G + 7) // 8 * 8
    slots = ng * _G
    padcnt = slots - ept

    src2 = edge_index[0].reshape(_TILES, ept)
    dst2 = edge_index[1].reshape(_TILES, ept)
    padv = (n + (jnp.arange(padcnt, dtype=jnp.int32) % (npad - n)))
    padv = jnp.broadcast_to(padv, (_TILES, padcnt))
    src3 = jnp.concatenate([src2, padv], axis=1).reshape(_TILES, ng, _G)
    dst3 = jnp.concatenate([dst2, padv], axis=1).reshape(_TILES, ng, _G)

    zerosd = jnp.zeros((128, d), jnp.float32)
    onesd = jnp.ones((_G, _DEGW), jnp.float32)
    zerosw = jnp.zeros((_G, _DEGW), jnp.float32)
    xpad = jnp.zeros((npad, d), features.dtype).at[:n].set(features)

    agg = _make_agg_kernel(npad, d, ng)
    deg = _make_deg_kernel(npad, ng)

    degi_p = deg(dst3, onesd, zerosw)
    dego_p = deg(src3, onesd, zerosw)
    degi = degi_p[0, :, :1] + degi_p[1, :, :1]
    dego = dego_p[0, :, :1] + dego_p[1, :, :1]
    mm_first = pl.pallas_call(
        _mm_first_body, out_shape=jax.ShapeDtypeStruct((npad, d), jnp.float32))
    mm_mid = pl.pallas_call(
        _mm_mid_body, out_shape=jax.ShapeDtypeStruct((npad, d), jnp.float32))
    readout = pl.pallas_call(
        _make_readout_body(n),
        out_shape=jax.ShapeDtypeStruct((1, Wp.shape[1]), jnp.float32))

    b1r, b2r, b3r = b1.reshape(1, d), b2.reshape(1, d), b3.reshape(1, d)
    bpr = bp.reshape(1, -1)

    z1 = mm_first(xpad, W1, dego)
    p1 = agg(z1, src3, dst3, zerosd)
    z2 = mm_mid(p1, degi, b1r, W2, dego)
    p2 = agg(z2, src3, dst3, zerosd)
    z3 = mm_mid(p2, degi, b2r, W3, dego)
    p3 = agg(z3, src3, dst3, zerosd)
    return readout(p3, degi, b3r, Wp, bpr)


# speedup vs baseline: 10.0720x; 1.0277x over previous
import functools

import jax
import jax.numpy as jnp
from jax import lax
from jax.experimental import pallas as pl
from jax.experimental.pallas import tpu as pltpu
from jax.experimental.pallas import tpu_sc as plsc

_NC = 2
_NS = 16
_TILES = _NC * _NS
_DEGW = 16
_G = 128


def _inv_sqrt(d):
    safe = jnp.where(d > 0, d, 1.0)
    return jnp.where(d > 0, lax.rsqrt(safe), 0.0)


def _mesh():
    return plsc.VectorSubcoreMesh(core_axis_name="c", subcore_axis_name="s",
                                  num_cores=_NC, num_subcores=_NS)


def _make_agg_kernel(npad, d, ng):
    rows_per_tile = npad // _NS
    zchunk = _G
    nz = rows_per_tile // zchunk

    cg = 16
    assert ng % cg == 0

    @functools.partial(
        pl.kernel,
        mesh=_mesh(),
        out_type=jax.ShapeDtypeStruct((_NC, npad, d), jnp.float32),
        scratch_types=[
            pltpu.VMEM_SHARED((npad, d), jnp.float32),
            pltpu.VMEM((cg, _G), jnp.int32),
            pltpu.VMEM((cg, _G), jnp.int32),
            pltpu.VMEM((_G, d), jnp.float32),
            pltpu.VMEM((_G, d), jnp.float32),
            pltpu.SemaphoreType.DMA,
            pltpu.SemaphoreType.DMA,
            pltpu.SemaphoreType.DMA,
            pltpu.SemaphoreType.DMA,
        ],
    )
    def agg_kernel(z_hbm, src_hbm, dst_hbm, zeros_hbm, out_hbm,
                   acc, idx_s, idx_d, rows_a, rows_b,
                   gsem_a, gsem_b, ssem_a, ssem_b):
        c = lax.axis_index("c")
        s = lax.axis_index("s")
        t = c * _NS + s
        base = s * rows_per_tile
        pltpu.sync_copy(zeros_hbm, rows_a)
        for k in range(nz):
            pltpu.sync_copy(rows_a, acc.at[pl.ds(base + k * zchunk, zchunk)])
        plsc.subcore_barrier()

        bufs = (rows_a, rows_b)
        gsems = (gsem_a, gsem_b)
        ssems = (ssem_a, ssem_b)

        def chunk_body(cc, carry):
            pltpu.sync_copy(src_hbm.at[t, pl.ds(cc * cg, cg)], idx_s)
            pltpu.sync_copy(dst_hbm.at[t, pl.ds(cc * cg, cg)], idx_d)
            pend_g = [pltpu.async_copy(z_hbm.at[idx_s.at[0]], bufs[0],
                                       gsems[0]), None]
            pend_s = [None, None]
            for j in range(cg):
                p = j % 2
                q = 1 - p
                pend_g[p].wait()
                if pend_s[q] is not None:
                    pend_s[q].wait()
                if j + 1 < cg:
                    pend_g[q] = pltpu.async_copy(z_hbm.at[idx_s.at[j + 1]],
                                                 bufs[q], gsems[q])
                pend_s[p] = pltpu.async_copy(bufs[p], acc.at[idx_d.at[j]],
                                             ssems[p], add=True)
            pend_s[(cg - 1) % 2].wait()
            return carry

        lax.fori_loop(0, ng // cg, chunk_body, 0)
        plsc.subcore_barrier()
        for k in range(nz):
            sl = pl.ds(base + k * zchunk, zchunk)
            pltpu.sync_copy(acc.at[sl], rows_a)
            pltpu.sync_copy(rows_a, out_hbm.at[c, sl])

    return agg_kernel


def _make_deg_kernel(npad, ng):
    rows_per_tile = npad // _NS
    zchunk = _G
    nz = rows_per_tile // zchunk
    cg = 16
    assert ng % cg == 0

    @functools.partial(
        pl.kernel,
        mesh=_mesh(),
        out_type=jax.ShapeDtypeStruct((_NC, npad, _DEGW), jnp.float32),
        compiler_params=pltpu.CompilerParams(use_tc_tiling_on_sc=False),
        scratch_types=[
            pltpu.VMEM_SHARED((npad, _DEGW), jnp.float32),
            pltpu.VMEM((cg, _G), jnp.int32),
            pltpu.VMEM((_G, _DEGW), jnp.float32),
            pltpu.VMEM((_G, _DEGW), jnp.float32),
            pltpu.SemaphoreType.DMA,
        ],
    )
    def deg_kernel(ids_hbm, ones_hbm, zeros_hbm, out_hbm,
                   acc, idx_v, ones_v, stage, sem):
        c = lax.axis_index("c")
        s = lax.axis_index("s")
        t = c * _NS + s
        base = s * rows_per_tile
        pltpu.sync_copy(ones_hbm, ones_v)
        pltpu.sync_copy(zeros_hbm, stage)
        for k in range(nz):
            pltpu.sync_copy(stage, acc.at[pl.ds(base + k * zchunk, zchunk)])
        plsc.subcore_barrier()

        def chunk_body(cc, carry):
            pltpu.sync_copy(ids_hbm.at[t, pl.ds(cc * cg, cg)], idx_v)
            pends = [pltpu.async_copy(ones_v, acc.at[idx_v.at[j]], sem,
                                      add=True) for j in range(cg)]
            for p in pends:
                p.wait()
            return carry

        lax.fori_loop(0, ng // cg, chunk_body, 0)
        plsc.subcore_barrier()
        for k in range(nz):
            sl = pl.ds(base + k * zchunk, zchunk)
            pltpu.sync_copy(acc.at[sl], stage)
            pltpu.sync_copy(stage, out_hbm.at[c, sl])

    return deg_kernel


def _mm_first_body(x_ref, w_ref, dego_ref, o_ref):
    z = jnp.dot(x_ref[...], w_ref[...], preferred_element_type=jnp.float32)
    o_ref[...] = z * _inv_sqrt(dego_ref[...])


def _mm_mid_body(p_ref, degi_ref, b_ref, w_ref, dego_ref, o_ref):
    h = (p_ref[0] + p_ref[1]) * _inv_sqrt(degi_ref[...]) + b_ref[...]
    h = jnp.maximum(h, 0.0)
    z = jnp.dot(h, w_ref[...], preferred_element_type=jnp.float32)
    o_ref[...] = z * _inv_sqrt(dego_ref[...])


def _make_readout_body(n_real):
    def _readout_body(p_ref, degi_ref, b_ref, wp_ref, bp_ref, o_ref):
        npad = p_ref.shape[1]
        h = (p_ref[0] + p_ref[1]) * _inv_sqrt(degi_ref[...]) + b_ref[...]
        h = jnp.maximum(h, 0.0)
        row = lax.broadcasted_iota(jnp.int32, (npad, 1), 0)
        h = jnp.where(row < n_real, h, 0.0)
        r = jnp.sum(h, axis=0, keepdims=True) * (1.0 / n_real)
        o_ref[...] = jnp.dot(r, wp_ref[...],
                             preferred_element_type=jnp.float32) + bp_ref[...]
    return _readout_body


def kernel(features, edge_index, W1, b1, W2, b2, W3, b3, Wp, bp):
    n, d = features.shape
    e = edge_index.shape[1]
    npad = ((n + _NS * 128 - 1) // (_NS * 128)) * (_NS * 128)
    ept = e // _TILES
    assert ept * _TILES == e
    ng = ((ept + _G - 1) // _G + 7) // 8 * 8
    slots = ng * _G
    padcnt = slots - ept

    src2 = edge_index[0].reshape(_TILES, ept)
    dst2 = edge_index[1].reshape(_TILES, ept)
    padv = (n + (jnp.arange(padcnt, dtype=jnp.int32) % (npad - n)))
    padv = jnp.broadcast_to(padv, (_TILES, padcnt))
    src3 = jnp.concatenate([src2, padv], axis=1).reshape(_TILES, ng, _G)
    dst3 = jnp.concatenate([dst2, padv], axis=1).reshape(_TILES, ng, _G)

    zerosd = jnp.zeros((128, d), jnp.float32)
    onesd = jnp.ones((_G, _DEGW), jnp.float32)
    zerosw = jnp.zeros((_G, _DEGW), jnp.float32)
    xpad = jnp.zeros((npad, d), features.dtype).at[:n].set(features)

    agg = _make_agg_kernel(npad, d, ng)
    deg = _make_deg_kernel(npad, ng)

    degi_p = deg(dst3, onesd, zerosw)
    dego_p = deg(src3, onesd, zerosw)
    degi = degi_p[0, :, :1] + degi_p[1, :, :1]
    dego = dego_p[0, :, :1] + dego_p[1, :, :1]
    mm_first = pl.pallas_call(
        _mm_first_body, out_shape=jax.ShapeDtypeStruct((npad, d), jnp.float32))
    mm_mid = pl.pallas_call(
        _mm_mid_body, out_shape=jax.ShapeDtypeStruct((npad, d), jnp.float32))
    readout = pl.pallas_call(
        _make_readout_body(n),
        out_shape=jax.ShapeDtypeStruct((1, Wp.shape[1]), jnp.float32))

    b1r, b2r, b3r = b1.reshape(1, d), b2.reshape(1, d), b3.reshape(1, d)
    bpr = bp.reshape(1, -1)

    z1 = mm_first(xpad, W1, dego)
    p1 = agg(z1, src3, dst3, zerosd)
    z2 = mm_mid(p1, degi, b1r, W2, dego)
    p2 = agg(z2, src3, dst3, zerosd)
    z3 = mm_mid(p2, degi, b2r, W3, dego)
    p3 = agg(z3, src3, dst3, zerosd)
    return readout(p3, degi, b3r, Wp, bpr)


# speedup vs baseline: 10.1142x; 1.0042x over previous
import functools

import jax
import jax.numpy as jnp
from jax import lax
from jax.experimental import pallas as pl
from jax.experimental.pallas import tpu as pltpu
from jax.experimental.pallas import tpu_sc as plsc

_NC = 2
_NS = 16
_TILES = _NC * _NS
_DEGW = 16
_G = 128


def _inv_sqrt(d):
    safe = jnp.where(d > 0, d, 1.0)
    return jnp.where(d > 0, lax.rsqrt(safe), 0.0)


def _mesh():
    return plsc.VectorSubcoreMesh(core_axis_name="c", subcore_axis_name="s",
                                  num_cores=_NC, num_subcores=_NS)


def _make_agg_kernel(npad, d, ng):
    rows_per_tile = npad // _NS
    zchunk = _G
    nz = rows_per_tile // zchunk

    cg = 16
    assert ng % cg == 0

    @functools.partial(
        pl.kernel,
        mesh=_mesh(),
        out_type=jax.ShapeDtypeStruct((_NC, npad, d), jnp.float32),
        scratch_types=[
            pltpu.VMEM_SHARED((npad, d), jnp.float32),
            pltpu.VMEM((cg, _G), jnp.int32),
            pltpu.VMEM((cg, _G), jnp.int32),
            pltpu.VMEM((_G, d), jnp.float32),
            pltpu.VMEM((_G, d), jnp.float32),
            pltpu.SemaphoreType.DMA,
            pltpu.SemaphoreType.DMA,
            pltpu.SemaphoreType.DMA,
            pltpu.SemaphoreType.DMA,
        ],
    )
    def agg_kernel(z_hbm, src_hbm, dst_hbm, zeros_hbm, out_hbm,
                   acc, idx_s, idx_d, rows_a, rows_b,
                   gsem_a, gsem_b, ssem_a, ssem_b):
        c = lax.axis_index("c")
        s = lax.axis_index("s")
        t = c * _NS + s
        base = s * rows_per_tile
        pltpu.sync_copy(zeros_hbm, rows_a)
        for k in range(nz):
            pltpu.sync_copy(rows_a, acc.at[pl.ds(base + k * zchunk, zchunk)])
        plsc.subcore_barrier()

        bufs = (rows_a, rows_b)
        gsems = (gsem_a, gsem_b)
        ssems = (ssem_a, ssem_b)

        def chunk_body(cc, carry):
            pltpu.sync_copy(src_hbm.at[t, pl.ds(cc * cg, cg)], idx_s)
            pltpu.sync_copy(dst_hbm.at[t, pl.ds(cc * cg, cg)], idx_d)
            pend_g = [pltpu.async_copy(z_hbm.at[idx_s.at[0]], bufs[0],
                                       gsems[0]), None]
            pend_s = [None, None]
            for j in range(cg):
                p = j % 2
                q = 1 - p
                pend_g[p].wait()
                if pend_s[q] is not None:
                    pend_s[q].wait()
                if j + 1 < cg:
                    pend_g[q] = pltpu.async_copy(z_hbm.at[idx_s.at[j + 1]],
                                                 bufs[q], gsems[q])
                pend_s[p] = pltpu.async_copy(bufs[p], acc.at[idx_d.at[j]],
                                             ssems[p], add=True)
            pend_s[(cg - 1) % 2].wait()
            return carry

        lax.fori_loop(0, ng // cg, chunk_body, 0)
        plsc.subcore_barrier()
        for k in range(nz):
            sl = pl.ds(base + k * zchunk, zchunk)
            pltpu.sync_copy(acc.at[sl], rows_a)
            pltpu.sync_copy(rows_a, out_hbm.at[c, sl])

    return agg_kernel


def _make_deg_kernel(npad, ng):
    rows_per_tile = npad // _NS
    zchunk = _G
    nz = rows_per_tile // zchunk
    cg = 16
    assert ng % cg == 0

    @functools.partial(
        pl.kernel,
        mesh=_mesh(),
        out_type=jax.ShapeDtypeStruct((_NC, 2, npad, _DEGW), jnp.float32),
        compiler_params=pltpu.CompilerParams(use_tc_tiling_on_sc=False),
        scratch_types=[
            pltpu.VMEM_SHARED((npad, _DEGW), jnp.float32),
            pltpu.VMEM_SHARED((npad, _DEGW), jnp.float32),
            pltpu.VMEM((cg, _G), jnp.int32),
            pltpu.VMEM((cg, _G), jnp.int32),
            pltpu.VMEM((_G, _DEGW), jnp.float32),
            pltpu.VMEM((_G, _DEGW), jnp.float32),
            pltpu.SemaphoreType.DMA,
        ],
    )
    def deg_kernel(src_hbm, dst_hbm, ones_hbm, zeros_hbm, out_hbm,
                   acc_o, acc_i, idx_s, idx_d, ones_v, stage, sem):
        c = lax.axis_index("c")
        s = lax.axis_index("s")
        t = c * _NS + s
        base = s * rows_per_tile
        pltpu.sync_copy(ones_hbm, ones_v)
        pltpu.sync_copy(zeros_hbm, stage)
        for k in range(nz):
            pltpu.sync_copy(stage, acc_o.at[pl.ds(base + k * zchunk, zchunk)])
            pltpu.sync_copy(stage, acc_i.at[pl.ds(base + k * zchunk, zchunk)])
        plsc.subcore_barrier()

        def chunk_body(cc, carry):
            pltpu.sync_copy(src_hbm.at[t, pl.ds(cc * cg, cg)], idx_s)
            pltpu.sync_copy(dst_hbm.at[t, pl.ds(cc * cg, cg)], idx_d)
            pends = []
            for j in range(cg):
                pends.append(pltpu.async_copy(
                    ones_v, acc_o.at[idx_s.at[j]], sem, add=True))
                pends.append(pltpu.async_copy(
                    ones_v, acc_i.at[idx_d.at[j]], sem, add=True))
            for p in pends:
                p.wait()
            return carry

        lax.fori_loop(0, ng // cg, chunk_body, 0)
        plsc.subcore_barrier()
        for k in range(nz):
            sl = pl.ds(base + k * zchunk, zchunk)
            pltpu.sync_copy(acc_o.at[sl], stage)
            pltpu.sync_copy(stage, out_hbm.at[c, 0, sl])
            pltpu.sync_copy(acc_i.at[sl], stage)
            pltpu.sync_copy(stage, out_hbm.at[c, 1, sl])

    return deg_kernel


def _mm_first_body(x_ref, w_ref, dego_ref, o_ref):
    z = jnp.dot(x_ref[...], w_ref[...], preferred_element_type=jnp.float32)
    o_ref[...] = z * _inv_sqrt(dego_ref[...])


def _mm_mid_body(p_ref, degi_ref, b_ref, w_ref, dego_ref, o_ref):
    h = (p_ref[0] + p_ref[1]) * _inv_sqrt(degi_ref[...]) + b_ref[...]
    h = jnp.maximum(h, 0.0)
    z = jnp.dot(h, w_ref[...], preferred_element_type=jnp.float32)
    o_ref[...] = z * _inv_sqrt(dego_ref[...])


def _make_readout_body(n_real):
    def _readout_body(p_ref, degi_ref, b_ref, wp_ref, bp_ref, o_ref):
        npad = p_ref.shape[1]
        h = (p_ref[0] + p_ref[1]) * _inv_sqrt(degi_ref[...]) + b_ref[...]
        h = jnp.maximum(h, 0.0)
        row = lax.broadcasted_iota(jnp.int32, (npad, 1), 0)
        h = jnp.where(row < n_real, h, 0.0)
        r = jnp.sum(h, axis=0, keepdims=True) * (1.0 / n_real)
        o_ref[...] = jnp.dot(r, wp_ref[...],
                             preferred_element_type=jnp.float32) + bp_ref[...]
    return _readout_body


def kernel(features, edge_index, W1, b1, W2, b2, W3, b3, Wp, bp):
    n, d = features.shape
    e = edge_index.shape[1]
    npad = ((n + _NS * 128 - 1) // (_NS * 128)) * (_NS * 128)
    ept = e // _TILES
    assert ept * _TILES == e
    ng = ((ept + _G - 1) // _G + 7) // 8 * 8
    slots = ng * _G
    padcnt = slots - ept

    src2 = edge_index[0].reshape(_TILES, ept)
    dst2 = edge_index[1].reshape(_TILES, ept)
    padv = (n + (jnp.arange(padcnt, dtype=jnp.int32) % (npad - n)))
    padv = jnp.broadcast_to(padv, (_TILES, padcnt))
    src3 = jnp.concatenate([src2, padv], axis=1).reshape(_TILES, ng, _G)
    dst3 = jnp.concatenate([dst2, padv], axis=1).reshape(_TILES, ng, _G)

    zerosd = jnp.zeros((128, d), jnp.float32)
    onesd = jnp.ones((_G, _DEGW), jnp.float32)
    zerosw = jnp.zeros((_G, _DEGW), jnp.float32)
    xpad = jnp.zeros((npad, d), features.dtype).at[:n].set(features)

    agg = _make_agg_kernel(npad, d, ng)
    deg = _make_deg_kernel(npad, ng)

    degp = deg(src3, dst3, onesd, zerosw)
    dego = degp[0, 0, :, :1] + degp[1, 0, :, :1]
    degi = degp[0, 1, :, :1] + degp[1, 1, :, :1]
    mm_first = pl.pallas_call(
        _mm_first_body, out_shape=jax.ShapeDtypeStruct((npad, d), jnp.float32))
    mm_mid = pl.pallas_call(
        _mm_mid_body, out_shape=jax.ShapeDtypeStruct((npad, d), jnp.float32))
    readout = pl.pallas_call(
        _make_readout_body(n),
        out_shape=jax.ShapeDtypeStruct((1, Wp.shape[1]), jnp.float32))

    b1r, b2r, b3r = b1.reshape(1, d), b2.reshape(1, d), b3.reshape(1, d)
    bpr = bp.reshape(1, -1)

    z1 = mm_first(xpad, W1, dego)
    p1 = agg(z1, src3, dst3, zerosd)
    z2 = mm_mid(p1, degi, b1r, W2, dego)
    p2 = agg(z2, src3, dst3, zerosd)
    z3 = mm_mid(p2, degi, b2r, W3, dego)
    p3 = agg(z3, src3, dst3, zerosd)
    return readout(p3, degi, b3r, Wp, bpr)
